# baseline (speedup 1.0000x reference)
import dataclasses
import functools

import jax
import jax.numpy as jnp
from jax import lax
from jax.experimental import pallas as pl
from jax.experimental.pallas import tpu as pltpu
from jax.experimental.pallas import tpu_sc as plsc

N = 10000
E = 320000
F = 128
C = 40
NC, NS = 2, 16
NW = NC * NS
EB = 80
NBLK = E // (NW * EB)
RPT = 624
TAIL0 = NS * RPT
TAILN = N - TAIL0
RB = 1000
BN_SCALE = float(1.0 / (1.0 + 1e-5) ** 0.5)

_mesh = plsc.VectorSubcoreMesh(core_axis_name="c", subcore_axis_name="s")
_CP = pltpu.CompilerParams()
if "needs_layout_passes" in pltpu.CompilerParams.__dataclass_fields__:
    _CP = dataclasses.replace(_CP, needs_layout_passes=False)


def _make_prop(d):
    @functools.partial(
        pl.kernel,
        out_type=jax.ShapeDtypeStruct((NC, N, d), jnp.float32),
        mesh=_mesh,
        scratch_types=[
            [pltpu.VMEM((EB,), jnp.int32) for _ in range(4)],
            pltpu.VMEM((NBLK, EB), jnp.int32),
            pltpu.VMEM((EB, d), jnp.float32),
            pltpu.VMEM((EB, d), jnp.float32),
            pltpu.VMEM_SHARED((N, d), jnp.float32),
            [pltpu.SemaphoreType.DMA for _ in range(4)],
            pltpu.SemaphoreType.DMA,
            pltpu.SemaphoreType.DMA,
            pltpu.SemaphoreType.DMA,
            pltpu.SemaphoreType.DMA,
        ],
    )
    def prop(ht_hbm, src_hbm, dst_hbm, zeros_hbm, out_hbm,
             sv, dstv, rows0, rows1, acc, si, sg0, sg1, ss0, ss1):
        cid = lax.axis_index("c")
        sid = lax.axis_index("s")
        wid = sid * NC + cid
        row0 = sid * RPT
        pltpu.sync_copy(zeros_hbm, acc.at[pl.ds(row0, RPT)])

        @pl.when(sid == NS - 1)
        def _():
            pltpu.sync_copy(zeros_hbm.at[pl.ds(0, TAILN)],
                            acc.at[pl.ds(TAIL0, TAILN)])

        my_src = src_hbm.at[wid]
        pltpu.sync_copy(dst_hbm.at[wid], dstv)
        plsc.subcore_barrier()

        def i_start(j, v):
            pltpu.async_copy(my_src.at[jnp.minimum(j, NBLK - 1)], sv[v],
                             si[v])

        def i_wait(v):
            pltpu.make_async_copy(my_src.at[0], sv[v], si[v]).wait()

        def g_start(v, buf, sem):
            pltpu.async_copy(ht_hbm.at[sv[v]], buf, sem)

        def g_wait(buf, sem):
            pltpu.make_async_copy(ht_hbm.at[sv[0]], buf, sem).wait()

        def s_start(j, buf, sem):
            pltpu.async_copy(buf, acc.at[dstv.at[j]], sem, add=True)

        def s_wait(buf, sem):
            pltpu.make_async_copy(buf, acc.at[dstv.at[0]], sem).wait()

        pltpu.sync_copy(zeros_hbm.at[pl.ds(0, EB)], rows1)
        for v in range(4):
            i_start(v, v)
        i_wait(0)
        g_start(0, rows0, sg0)
        s_start(0, rows1, ss1)

        @pl.loop(0, (NBLK - 1) // 4)
        def _(k):
            j0 = 4 * k
            g_wait(rows0, sg0)
            s_start(j0, rows0, ss0)
            s_wait(rows1, ss1)
            i_wait(1)
            g_start(1, rows1, sg1)
            i_start(j0 + 4, 0)
            g_wait(rows1, sg1)
            s_start(j0 + 1, rows1, ss1)
            s_wait(rows0, ss0)
            i_wait(2)
            g_start(2, rows0, sg0)
            i_start(j0 + 5, 1)
            g_wait(rows0, sg0)
            s_start(j0 + 2, rows0, ss0)
            s_wait(rows1, ss1)
            i_wait(3)
            g_start(3, rows1, sg1)
            i_start(j0 + 6, 2)
            g_wait(rows1, sg1)
            s_start(j0 + 3, rows1, ss1)
            s_wait(rows0, ss0)
            i_wait(0)
            g_start(0, rows0, sg0)
            i_start(j0 + 7, 3)

        g_wait(rows0, sg0)
        s_start(NBLK - 1, rows0, ss0)
        s_wait(rows1, ss1)
        s_wait(rows0, ss0)
        i_wait(1)
        i_wait(2)
        i_wait(3)
        plsc.subcore_barrier()
        pltpu.sync_copy(acc.at[pl.ds(row0, RPT)],
                        out_hbm.at[cid, pl.ds(row0, RPT)])

        @pl.when(sid == NS - 1)
        def _():
            pltpu.sync_copy(acc.at[pl.ds(TAIL0, TAILN)],
                            out_hbm.at[cid, pl.ds(TAIL0, TAILN)])

    return prop


_prop128 = _make_prop(F)


@functools.partial(
    pl.kernel,
    out_type=jax.ShapeDtypeStruct((NC * N,), jnp.float32),
    mesh=_mesh,
    scratch_types=[
        pltpu.VMEM((E // NW // 16, 16), jnp.int32),
        pltpu.VMEM((N,), jnp.float32),
        pltpu.VMEM((NS * (RPT + 16),), jnp.float32),
        pltpu.VMEM((RPT + 16,), jnp.float32),
        pltpu.VMEM_SHARED((NS * N,), jnp.float32),
    ],
    compiler_params=_CP,
)
def _deg_kernel(dst_hbm, zeros_hbm, out_hbm, dstv, hist, tmp, res, stage):
    cid = lax.axis_index("c")
    sid = lax.axis_index("s")
    wid = sid * NC + cid
    pltpu.sync_copy(zeros_hbm, hist)
    pltpu.sync_copy(dst_hbm.at[wid], dstv)
    ones = jnp.ones((16,), jnp.float32)

    @pl.loop(0, E // NW // 16)
    def _(r):
        plsc.addupdate_scatter(hist, [dstv[r]], ones)

    pltpu.sync_copy(hist, stage.at[pl.ds(sid * N, N)])
    plsc.subcore_barrier()

    c0 = sid * RPT
    W16 = RPT + 16

    @pl.loop(0, NS)
    def _(t):
        pltpu.sync_copy(stage.at[pl.ds(t * N + c0, RPT)],
                        tmp.at[pl.ds(t * W16, RPT)])

    @pl.when(sid == NS - 1)
    def _():
        @pl.loop(0, NS)
        def _(t):
            pltpu.sync_copy(stage.at[pl.ds(t * N + TAIL0, TAILN)],
                            tmp.at[pl.ds(t * W16 + RPT, TAILN)])

    def reduce_chunk(ch):
        acc = tmp[pl.ds(ch * 16, 16)]
        for t in range(1, NS):
            acc = acc + tmp[pl.ds(t * W16 + ch * 16, 16)]
        res[pl.ds(ch * 16, 16)] = acc

    @pl.loop(0, RPT // 16)
    def _(ch):
        reduce_chunk(ch)

    @pl.when(sid == NS - 1)
    def _():
        reduce_chunk(RPT // 16)

    pltpu.sync_copy(res.at[pl.ds(0, RPT)],
                    out_hbm.at[pl.ds(cid * N + c0, RPT)])

    @pl.when(sid == NS - 1)
    def _():
        pltpu.sync_copy(res.at[pl.ds(RPT, TAILN)],
                        out_hbm.at[pl.ds(cid * N + TAIL0, TAILN)])


_DEG_SPEC = pl.BlockSpec((NC, RB, 1), lambda i: (0, i, 0))
_DIS_SPEC = pl.BlockSpec((RB, 1), lambda i: (i, 0))


def _mm_body(x_ref, w_ref, o_ref):
    o_ref[...] = jnp.dot(x_ref[...], w_ref[...],
                         preferred_element_type=jnp.float32,
                         precision=lax.Precision.HIGHEST)


_matmul = pl.pallas_call(
    _mm_body,
    grid=(N // RB,),
    in_specs=[pl.BlockSpec((RB, F), lambda i: (i, 0)),
              pl.BlockSpec((F, F), lambda i: (0, 0))],
    out_specs=pl.BlockSpec((RB, F), lambda i: (i, 0)),
    out_shape=jax.ShapeDtypeStruct((N, F), jnp.float32),
)


def _scale_body(xw_ref, deg_ref, o_ref, dis_ref):
    d = deg_ref[0] + deg_ref[1] + 1.0
    dis = lax.rsqrt(d)
    o_ref[...] = xw_ref[...] * dis
    dis_ref[...] = dis


_scale = pl.pallas_call(
    _scale_body,
    grid=(N // RB,),
    in_specs=[pl.BlockSpec((RB, F), lambda i: (i, 0)), _DEG_SPEC],
    out_specs=[pl.BlockSpec((RB, F), lambda i: (i, 0)),
               pl.BlockSpec((RB, 1), lambda i: (i, 0))],
    out_shape=[jax.ShapeDtypeStruct((N, F), jnp.float32),
               jax.ShapeDtypeStruct((N, 1), jnp.float32)],
)


def _stage_body(p_ref, ht_ref, dis_ref, b_ref, g_ref, be_ref, w_ref, o_ref):
    dis = dis_ref[...]
    agg = dis * (p_ref[0] + p_ref[1] + ht_ref[...]) + b_ref[...]
    h = jnp.maximum(g_ref[...] * (agg * BN_SCALE) + be_ref[...], 0.0)
    o_ref[...] = jnp.dot(h, w_ref[...],
                         preferred_element_type=jnp.float32,
                         precision=lax.Precision.HIGHEST) * dis


def _make_stage(d_out):
    return pl.pallas_call(
        _stage_body,
        grid=(N // RB,),
        in_specs=[
            pl.BlockSpec((NC, RB, F), lambda i: (0, i, 0)),
            pl.BlockSpec((RB, F), lambda i: (i, 0)),
            _DIS_SPEC,
            pl.BlockSpec((F,), lambda i: (0,)),
            pl.BlockSpec((F,), lambda i: (0,)),
            pl.BlockSpec((F,), lambda i: (0,)),
            pl.BlockSpec((F, d_out), lambda i: (0, 0)),
        ],
        out_specs=pl.BlockSpec((RB, d_out), lambda i: (i, 0)),
        out_shape=jax.ShapeDtypeStruct((N, d_out), jnp.float32),
    )


_stage128 = _make_stage(F)


def _relu_scale_body(p_ref, ht_ref, dis_ref, b_ref, g_ref, be_ref, o_ref):
    dis = dis_ref[...]
    agg = dis * (p_ref[0] + p_ref[1] + ht_ref[...]) + b_ref[...]
    h = jnp.maximum(g_ref[...] * (agg * BN_SCALE) + be_ref[...], 0.0)
    o_ref[...] = h * dis


_relu_scale = pl.pallas_call(
    _relu_scale_body,
    grid=(N // RB,),
    in_specs=[
        pl.BlockSpec((NC, RB, F), lambda i: (0, i, 0)),
        pl.BlockSpec((RB, F), lambda i: (i, 0)),
        _DIS_SPEC,
        pl.BlockSpec((F,), lambda i: (0,)),
        pl.BlockSpec((F,), lambda i: (0,)),
        pl.BlockSpec((F,), lambda i: (0,)),
    ],
    out_specs=pl.BlockSpec((RB, F), lambda i: (i, 0)),
    out_shape=jax.ShapeDtypeStruct((N, F), jnp.float32),
)


def _final_body(p_ref, ht_ref, dis_ref, w_ref, b_ref, o_ref):
    dis = dis_ref[...]
    h = dis * (p_ref[0] + p_ref[1] + ht_ref[...])
    logits = jnp.dot(h, w_ref[...],
                     preferred_element_type=jnp.float32,
                     precision=lax.Precision.HIGHEST) + b_ref[...]
    m = jnp.max(logits, axis=1, keepdims=True)
    s = jnp.sum(jnp.exp(logits - m), axis=1, keepdims=True)
    o_ref[...] = logits - m - jnp.log(s)


_final = pl.pallas_call(
    _final_body,
    grid=(N // RB,),
    in_specs=[
        pl.BlockSpec((NC, RB, F), lambda i: (0, i, 0)),
        pl.BlockSpec((RB, F), lambda i: (i, 0)),
        _DIS_SPEC,
        pl.BlockSpec((F, C), lambda i: (0, 0)),
        pl.BlockSpec((C,), lambda i: (0,)),
    ],
    out_specs=pl.BlockSpec((RB, C), lambda i: (i, 0)),
    out_shape=jax.ShapeDtypeStruct((N, C), jnp.float32),
)


def kernel(x, edge_index, W1, b1, g1, be1, W2, b2, g2, be2, W3, b3):
    src3 = edge_index[0].reshape(NW, NBLK, EB)
    dst3 = edge_index[1].reshape(NW, NBLK, EB)
    dsth = edge_index[1].reshape(NW, E // NW // 16, 16)
    zeros128 = jnp.zeros((RPT, F), jnp.float32)
    zerosN = jnp.zeros((N,), jnp.float32)

    deg = _deg_kernel(dsth, zerosN).reshape(NC, N, 1)
    xw = _matmul(x, W1)
    ht1, dis = _scale(xw, deg)
    p1 = _prop128(ht1, src3, dst3, zeros128)
    ht2 = _stage128(p1, ht1, dis, b1, g1, be1, W2)
    p2 = _prop128(ht2, src3, dst3, zeros128)
    h2t = _relu_scale(p2, ht2, dis, b2, g2, be2)
    p3 = _prop128(h2t, src3, dst3, zeros128)
    return _final(p3, h2t, dis, W3, b3)


# speedup vs baseline: 1.0318x; 1.0318x over previous
import dataclasses
import functools

import jax
import jax.numpy as jnp
from jax import lax
from jax.experimental import pallas as pl
from jax.experimental.pallas import tpu as pltpu
from jax.experimental.pallas import tpu_sc as plsc

N = 10000
E = 320000
F = 128
C = 40
NC, NS = 2, 16
NW = NC * NS
EB = 80
NBLK = E // (NW * EB)
RPT = 624
TAIL0 = NS * RPT
TAILN = N - TAIL0
RB = 2000
BN_SCALE = float(1.0 / (1.0 + 1e-5) ** 0.5)

_mesh = plsc.VectorSubcoreMesh(core_axis_name="c", subcore_axis_name="s")
_CP = pltpu.CompilerParams()
if "needs_layout_passes" in pltpu.CompilerParams.__dataclass_fields__:
    _CP = dataclasses.replace(_CP, needs_layout_passes=False)


def _make_prop(d):
    @functools.partial(
        pl.kernel,
        out_type=jax.ShapeDtypeStruct((NC, N, d), jnp.float32),
        mesh=_mesh,
        scratch_types=[
            [pltpu.VMEM((EB,), jnp.int32) for _ in range(4)],
            pltpu.VMEM((NBLK, EB), jnp.int32),
            pltpu.VMEM((EB, d), jnp.float32),
            pltpu.VMEM((EB, d), jnp.float32),
            pltpu.VMEM_SHARED((N, d), jnp.float32),
            [pltpu.SemaphoreType.DMA for _ in range(4)],
            pltpu.SemaphoreType.DMA,
            pltpu.SemaphoreType.DMA,
            pltpu.SemaphoreType.DMA,
            pltpu.SemaphoreType.DMA,
        ],
    )
    def prop(ht_hbm, src_hbm, dst_hbm, zeros_hbm, out_hbm,
             sv, dstv, rows0, rows1, acc, si, sg0, sg1, ss0, ss1):
        cid = lax.axis_index("c")
        sid = lax.axis_index("s")
        wid = sid * NC + cid
        row0 = sid * RPT
        pltpu.sync_copy(zeros_hbm, acc.at[pl.ds(row0, RPT)])

        @pl.when(sid == NS - 1)
        def _():
            pltpu.sync_copy(zeros_hbm.at[pl.ds(0, TAILN)],
                            acc.at[pl.ds(TAIL0, TAILN)])

        my_src = src_hbm.at[wid]
        pltpu.sync_copy(dst_hbm.at[wid], dstv)
        plsc.subcore_barrier()

        def i_start(j, v):
            pltpu.async_copy(my_src.at[jnp.minimum(j, NBLK - 1)], sv[v],
                             si[v])

        def i_wait(v):
            pltpu.make_async_copy(my_src.at[0], sv[v], si[v]).wait()

        def g_start(v, buf, sem):
            pltpu.async_copy(ht_hbm.at[sv[v]], buf, sem)

        def g_wait(buf, sem):
            pltpu.make_async_copy(ht_hbm.at[sv[0]], buf, sem).wait()

        def s_start(j, buf, sem):
            pltpu.async_copy(buf, acc.at[dstv.at[j]], sem, add=True)

        def s_wait(buf, sem):
            pltpu.make_async_copy(buf, acc.at[dstv.at[0]], sem).wait()

        for v in range(4):
            i_start(v, v)
        i_wait(0)
        g_start(0, rows0, sg0)

        @pl.loop(0, (NBLK - 1) // 4)
        def _(k):
            j0 = 4 * k
            g_wait(rows0, sg0)
            i_wait(1)
            g_start(1, rows1, sg1)
            s_start(j0, rows0, ss0)
            g_wait(rows1, sg1)
            s_wait(rows0, ss0)
            i_wait(2)
            g_start(2, rows0, sg0)
            i_start(j0 + 4, 0)
            s_start(j0 + 1, rows1, ss1)
            g_wait(rows0, sg0)
            s_wait(rows1, ss1)
            i_wait(3)
            g_start(3, rows1, sg1)
            i_start(j0 + 5, 1)
            s_start(j0 + 2, rows0, ss0)
            g_wait(rows1, sg1)
            s_wait(rows0, ss0)
            i_start(j0 + 6, 2)
            i_wait(0)
            g_start(0, rows0, sg0)
            i_start(j0 + 7, 3)
            s_start(j0 + 3, rows1, ss1)
            s_wait(rows1, ss1)

        g_wait(rows0, sg0)
        s_start(NBLK - 1, rows0, ss0)
        s_wait(rows0, ss0)
        i_wait(1)
        i_wait(2)
        i_wait(3)
        plsc.subcore_barrier()
        pltpu.sync_copy(acc.at[pl.ds(row0, RPT)],
                        out_hbm.at[cid, pl.ds(row0, RPT)])

        @pl.when(sid == NS - 1)
        def _():
            pltpu.sync_copy(acc.at[pl.ds(TAIL0, TAILN)],
                            out_hbm.at[cid, pl.ds(TAIL0, TAILN)])

    return prop


_prop128 = _make_prop(F)


@functools.partial(
    pl.kernel,
    out_type=jax.ShapeDtypeStruct((NC * N,), jnp.float32),
    mesh=_mesh,
    scratch_types=[
        pltpu.VMEM((E // NW // 16, 16), jnp.int32),
        pltpu.VMEM((N,), jnp.float32),
        pltpu.VMEM((NS * (RPT + 16),), jnp.float32),
        pltpu.VMEM((RPT + 16,), jnp.float32),
        pltpu.VMEM_SHARED((NS * N,), jnp.float32),
    ],
    compiler_params=_CP,
)
def _deg_kernel(dst_hbm, zeros_hbm, out_hbm, dstv, hist, tmp, res, stage):
    cid = lax.axis_index("c")
    sid = lax.axis_index("s")
    wid = sid * NC + cid
    pltpu.sync_copy(zeros_hbm, hist)
    pltpu.sync_copy(dst_hbm.at[wid], dstv)
    ones = jnp.ones((16,), jnp.float32)

    @pl.loop(0, E // NW // 16)
    def _(r):
        plsc.addupdate_scatter(hist, [dstv[r]], ones)

    pltpu.sync_copy(hist, stage.at[pl.ds(sid * N, N)])
    plsc.subcore_barrier()

    c0 = sid * RPT
    W16 = RPT + 16

    @pl.loop(0, NS)
    def _(t):
        pltpu.sync_copy(stage.at[pl.ds(t * N + c0, RPT)],
                        tmp.at[pl.ds(t * W16, RPT)])

    @pl.when(sid == NS - 1)
    def _():
        @pl.loop(0, NS)
        def _(t):
            pltpu.sync_copy(stage.at[pl.ds(t * N + TAIL0, TAILN)],
                            tmp.at[pl.ds(t * W16 + RPT, TAILN)])

    def reduce_chunk(ch):
        acc = tmp[pl.ds(ch * 16, 16)]
        for t in range(1, NS):
            acc = acc + tmp[pl.ds(t * W16 + ch * 16, 16)]
        res[pl.ds(ch * 16, 16)] = acc

    @pl.loop(0, RPT // 16)
    def _(ch):
        reduce_chunk(ch)

    @pl.when(sid == NS - 1)
    def _():
        reduce_chunk(RPT // 16)

    pltpu.sync_copy(res.at[pl.ds(0, RPT)],
                    out_hbm.at[pl.ds(cid * N + c0, RPT)])

    @pl.when(sid == NS - 1)
    def _():
        pltpu.sync_copy(res.at[pl.ds(RPT, TAILN)],
                        out_hbm.at[pl.ds(cid * N + TAIL0, TAILN)])


_DEG_SPEC = pl.BlockSpec((NC, RB, 1), lambda i: (0, i, 0))
_DIS_SPEC = pl.BlockSpec((RB, 1), lambda i: (i, 0))


def _mm_body(x_ref, w_ref, o_ref):
    o_ref[...] = jnp.dot(x_ref[...], w_ref[...],
                         preferred_element_type=jnp.float32,
                         precision=lax.Precision.HIGHEST)


_matmul = pl.pallas_call(
    _mm_body,
    grid=(N // RB,),
    in_specs=[pl.BlockSpec((RB, F), lambda i: (i, 0)),
              pl.BlockSpec((F, F), lambda i: (0, 0))],
    out_specs=pl.BlockSpec((RB, F), lambda i: (i, 0)),
    out_shape=jax.ShapeDtypeStruct((N, F), jnp.float32),
)


def _scale_body(xw_ref, deg_ref, o_ref, dis_ref):
    d = deg_ref[0] + deg_ref[1] + 1.0
    dis = lax.rsqrt(d)
    o_ref[...] = xw_ref[...] * dis
    dis_ref[...] = dis


_scale = pl.pallas_call(
    _scale_body,
    grid=(N // RB,),
    in_specs=[pl.BlockSpec((RB, F), lambda i: (i, 0)), _DEG_SPEC],
    out_specs=[pl.BlockSpec((RB, F), lambda i: (i, 0)),
               pl.BlockSpec((RB, 1), lambda i: (i, 0))],
    out_shape=[jax.ShapeDtypeStruct((N, F), jnp.float32),
               jax.ShapeDtypeStruct((N, 1), jnp.float32)],
)


def _stage_body(p_ref, ht_ref, dis_ref, b_ref, g_ref, be_ref, w_ref, o_ref):
    dis = dis_ref[...]
    agg = dis * (p_ref[0] + p_ref[1] + ht_ref[...]) + b_ref[...]
    h = jnp.maximum(g_ref[...] * (agg * BN_SCALE) + be_ref[...], 0.0)
    o_ref[...] = jnp.dot(h, w_ref[...],
                         preferred_element_type=jnp.float32,
                         precision=lax.Precision.HIGHEST) * dis


def _make_stage(d_out):
    return pl.pallas_call(
        _stage_body,
        grid=(N // RB,),
        in_specs=[
            pl.BlockSpec((NC, RB, F), lambda i: (0, i, 0)),
            pl.BlockSpec((RB, F), lambda i: (i, 0)),
            _DIS_SPEC,
            pl.BlockSpec((F,), lambda i: (0,)),
            pl.BlockSpec((F,), lambda i: (0,)),
            pl.BlockSpec((F,), lambda i: (0,)),
            pl.BlockSpec((F, d_out), lambda i: (0, 0)),
        ],
        out_specs=pl.BlockSpec((RB, d_out), lambda i: (i, 0)),
        out_shape=jax.ShapeDtypeStruct((N, d_out), jnp.float32),
    )


_stage128 = _make_stage(F)


def _relu_scale_body(p_ref, ht_ref, dis_ref, b_ref, g_ref, be_ref, o_ref):
    dis = dis_ref[...]
    agg = dis * (p_ref[0] + p_ref[1] + ht_ref[...]) + b_ref[...]
    h = jnp.maximum(g_ref[...] * (agg * BN_SCALE) + be_ref[...], 0.0)
    o_ref[...] = h * dis


_relu_scale = pl.pallas_call(
    _relu_scale_body,
    grid=(N // RB,),
    in_specs=[
        pl.BlockSpec((NC, RB, F), lambda i: (0, i, 0)),
        pl.BlockSpec((RB, F), lambda i: (i, 0)),
        _DIS_SPEC,
        pl.BlockSpec((F,), lambda i: (0,)),
        pl.BlockSpec((F,), lambda i: (0,)),
        pl.BlockSpec((F,), lambda i: (0,)),
    ],
    out_specs=pl.BlockSpec((RB, F), lambda i: (i, 0)),
    out_shape=jax.ShapeDtypeStruct((N, F), jnp.float32),
)


def _final_body(p_ref, ht_ref, dis_ref, w_ref, b_ref, o_ref):
    dis = dis_ref[...]
    h = dis * (p_ref[0] + p_ref[1] + ht_ref[...])
    logits = jnp.dot(h, w_ref[...],
                     preferred_element_type=jnp.float32,
                     precision=lax.Precision.HIGHEST) + b_ref[...]
    m = jnp.max(logits, axis=1, keepdims=True)
    s = jnp.sum(jnp.exp(logits - m), axis=1, keepdims=True)
    o_ref[...] = logits - m - jnp.log(s)


_final = pl.pallas_call(
    _final_body,
    grid=(N // RB,),
    in_specs=[
        pl.BlockSpec((NC, RB, F), lambda i: (0, i, 0)),
        pl.BlockSpec((RB, F), lambda i: (i, 0)),
        _DIS_SPEC,
        pl.BlockSpec((F, C), lambda i: (0, 0)),
        pl.BlockSpec((C,), lambda i: (0,)),
    ],
    out_specs=pl.BlockSpec((RB, C), lambda i: (i, 0)),
    out_shape=jax.ShapeDtypeStruct((N, C), jnp.float32),
)


def kernel(x, edge_index, W1, b1, g1, be1, W2, b2, g2, be2, W3, b3):
    src3 = edge_index[0].reshape(NW, NBLK, EB)
    dst3 = edge_index[1].reshape(NW, NBLK, EB)
    dsth = edge_index[1].reshape(NW, E // NW // 16, 16)
    zeros128 = jnp.zeros((RPT, F), jnp.float32)
    zerosN = jnp.zeros((N,), jnp.float32)

    deg = _deg_kernel(dsth, zerosN).reshape(NC, N, 1)
    xw = _matmul(x, W1)
    ht1, dis = _scale(xw, deg)
    p1 = _prop128(ht1, src3, dst3, zeros128)
    ht2 = _stage128(p1, ht1, dis, b1, g1, be1, W2)
    p2 = _prop128(ht2, src3, dst3, zeros128)
    h2t = _relu_scale(p2, ht2, dis, b2, g2, be2)
    p3 = _prop128(h2t, src3, dst3, zeros128)
    return _final(p3, h2t, dis, W3, b3)
